# baseline (speedup 1.0000x reference)
import jax
import jax.numpy as jnp
from jax import lax
from jax.experimental import pallas as pl
from jax.experimental.pallas import tpu as pltpu
from jax.experimental.pallas import tpu_sc as plsc

VOCAB_SIZE = 1000000
HDIM = 64
BATCH, SEQ = 4096, 200

_NW = 32
_VBLK = 128
_NFULL = VOCAB_SIZE // _VBLK
_BBLK = 128
_NBB = BATCH // _BBLK


def _iota16():
    return lax.iota(jnp.int32, 16)


def _worker_id():
    return lax.axis_index("s") * 2 + lax.axis_index("c")


def _format_body(wT_hbm, wtail_hbm, out_hbm, slab_v, pack_v, i_sems, w_sems):
    wid = _worker_id()
    niter = (_NFULL - wid + (_NW - 1)) // _NW
    iot = _iota16()

    def fire_in(i, buf):
        vb = (wid + i * _NW) * _VBLK
        pltpu.async_copy(
            wT_hbm.at[pl.ds(0, HDIM), pl.ds(vb, _VBLK)], slab_v.at[buf],
            i_sems.at[buf],
        )

    fire_in(0, 0)

    def body(i, carry):
        buf = i % 2
        nbuf = (i + 1) % 2

        @pl.when(i + 1 < niter)
        def _():
            fire_in(i + 1, nbuf)

        pltpu.make_async_copy(
            wT_hbm.at[pl.ds(0, HDIM), pl.ds(0, _VBLK)], slab_v.at[buf],
            i_sems.at[buf],
        ).wait()

        @pl.when(i >= 2)
        def _():
            pltpu.make_async_copy(
                pack_v.at[buf], out_hbm.at[pl.ds(0, HDIM)], w_sems.at[buf]
            ).wait()

        slab = slab_v.at[buf]
        pack = pack_v.at[buf]

        def trq(q, c2):
            for half in range(2):
                r = 2 * q + half
                rvec = jnp.full((16,), r, jnp.int32)
                for c0 in range(0, HDIM, 16):
                    v = plsc.load_gather(slab, [c0 + iot, rvec])
                    pack[q, pl.ds(half * HDIM + c0, 16)] = v
            return c2

        lax.fori_loop(0, HDIM, trq, 0, unroll=4)

        pltpu.async_copy(
            pack, out_hbm.at[pl.ds((wid + i * _NW) * (_VBLK // 2), HDIM)],
            w_sems.at[buf],
        )
        return carry

    lax.fori_loop(0, niter, body, 0, unroll=False)

    for buf in range(2):
        @pl.when(niter >= (2 - buf))
        def _():
            pltpu.make_async_copy(
                pack_v.at[buf], out_hbm.at[pl.ds(0, HDIM)], w_sems.at[buf]
            ).wait()

    @pl.when(wid == _NW - 1)
    def _():
        pltpu.sync_copy(wtail_hbm, slab_v.at[0])
        tail = slab_v.at[0]
        pack = pack_v.at[0]

        def tq(q, c2):
            for half in range(2):
                for c0 in range(0, HDIM, 16):
                    v = tail[2 * q + half, pl.ds(c0, 16)]
                    pack[q, pl.ds(half * HDIM + c0, 16)] = v
            return c2

        lax.fori_loop(0, 32, tq, 0, unroll=4)
        pltpu.sync_copy(
            pack_v.at[0, pl.ds(0, 32)],
            out_hbm.at[pl.ds(_NFULL * (_VBLK // 2), 32)],
        )


def _gather_body(idxT_hbm, table_hbm, out_hbm, idx_v, rows_v, trans_v,
                 g_sems, w_sems):
    wid = _worker_id()
    iot = _iota16()
    pltpu.sync_copy(idxT_hbm.at[pl.ds(0, SEQ), pl.ds(wid * _BBLK, _BBLK)],
                    idx_v)

    pltpu.async_copy(table_hbm.at[idx_v.at[0]], rows_v.at[0], g_sems.at[0])

    def body(j, carry):
        buf = j % 2
        nbuf = (j + 1) % 2

        @pl.when(j + 1 < SEQ)
        def _():
            pltpu.async_copy(
                table_hbm.at[idx_v.at[j + 1]], rows_v.at[nbuf], g_sems.at[nbuf]
            )

        pltpu.make_async_copy(
            table_hbm.at[idx_v.at[0]], rows_v.at[buf], g_sems.at[buf]
        ).wait()

        @pl.when(j >= 2)
        def _():
            pltpu.make_async_copy(
                trans_v.at[buf], out_hbm.at[0, pl.ds(0, 8), wid], w_sems.at[buf]
            ).wait()

        rows = rows_v.at[buf]
        trans = trans_v.at[buf]

        def trh(h, c2):
            hvec = jnp.full((16,), h, jnp.int32)
            for c0 in range(0, _BBLK, 16):
                v = plsc.load_gather(rows, [c0 + iot, hvec])
                trans[h // 8, h % 8, pl.ds(c0, 16)] = v
            return c2

        lax.fori_loop(0, HDIM, trh, 0, unroll=4)

        pltpu.async_copy(
            trans, out_hbm.at[j, pl.ds(0, 8), wid], w_sems.at[buf]
        )
        return carry

    lax.fori_loop(0, SEQ, body, 0, unroll=False)

    for buf in range(2):
        pltpu.make_async_copy(
            trans_v.at[buf], out_hbm.at[0, pl.ds(0, 8), wid], w_sems.at[buf]
        ).wait()


def _mesh():
    return plsc.VectorSubcoreMesh(core_axis_name="c", subcore_axis_name="s")


@jax.jit
def kernel(x, weight):
    wT = weight.T
    wtailp = jnp.pad(weight[VOCAB_SIZE - HDIM:], ((0, 0), (0, 128 - HDIM)))
    table2 = pl.kernel(
        _format_body,
        mesh=_mesh(),
        compiler_params=pltpu.CompilerParams(
            use_tc_tiling_on_sc=True, needs_layout_passes=False),
        out_type=jax.ShapeDtypeStruct((VOCAB_SIZE // 2, 128), jnp.float32),
        scratch_types=[
            pltpu.VMEM((2, HDIM, _VBLK), jnp.float32),
            pltpu.VMEM((2, HDIM, _VBLK), jnp.float32),
            pltpu.SemaphoreType.DMA((2,)),
            pltpu.SemaphoreType.DMA((2,)),
        ],
    )(wT, wtailp)
    table = table2.reshape(VOCAB_SIZE, HDIM)

    idxT = x.T.astype(jnp.int32)
    out5 = pl.kernel(
        _gather_body,
        mesh=_mesh(),
        compiler_params=pltpu.CompilerParams(
            use_tc_tiling_on_sc=False, needs_layout_passes=False),
        out_type=jax.ShapeDtypeStruct((SEQ, 8, _NBB, 8, _BBLK), jnp.float32),
        scratch_types=[
            pltpu.VMEM((SEQ, _BBLK), jnp.int32),
            pltpu.VMEM((2, _BBLK, HDIM), jnp.float32),
            pltpu.VMEM((2, 8, 8, _BBLK), jnp.float32),
            pltpu.SemaphoreType.DMA((2,)),
            pltpu.SemaphoreType.DMA((2,)),
        ],
    )(idxT, table)
    return out5.transpose(2, 4, 0, 1, 3).reshape(BATCH, SEQ, HDIM)


# speedup vs baseline: 2.9413x; 2.9413x over previous
import jax
import jax.numpy as jnp
from jax import lax
from jax.experimental import pallas as pl
from jax.experimental.pallas import tpu as pltpu
from jax.experimental.pallas import tpu_sc as plsc

VOCAB_SIZE = 1000000
HDIM = 64
B, L = 4096, 200

_PAD = 128
_NW = 32
_TOTAL = B * L
_PER_W = _TOTAL // _NW
_CHUNK = 128
_NCHUNK = _PER_W // _CHUNK
_NBUF = 8
_K = 4


def _embed_body(idx_hbm, table_hbm, out_hbm, idx_v, rows_v, g_sems, w_sems):
    wid = lax.axis_index("s") * 2 + lax.axis_index("c")
    base = wid * _PER_W
    pltpu.sync_copy(idx_hbm.at[pl.ds(wid * _NCHUNK, _NCHUNK)], idx_v)

    for b in range(_K):
        pltpu.async_copy(table_hbm.at[idx_v.at[b]], rows_v.at[b], g_sems.at[b])

    def body(j, carry):
        buf = j % _NBUF
        fbuf = (j + _K) % _NBUF

        @pl.when(j + _K < _NCHUNK)
        def _():
            @pl.when(j + _K >= _NBUF)
            def _():
                pltpu.make_async_copy(
                    rows_v.at[fbuf],
                    out_hbm.at[pl.ds(base, _CHUNK), pl.ds(0, HDIM)],
                    w_sems.at[fbuf],
                ).wait()

            pltpu.async_copy(
                table_hbm.at[idx_v.at[j + _K]], rows_v.at[fbuf], g_sems.at[fbuf]
            )

        pltpu.make_async_copy(
            table_hbm.at[idx_v.at[j]], rows_v.at[buf], g_sems.at[buf]
        ).wait()
        pltpu.async_copy(
            rows_v.at[buf],
            out_hbm.at[pl.ds(base + j * _CHUNK, _CHUNK), pl.ds(0, HDIM)],
            w_sems.at[buf],
        )
        return carry

    lax.fori_loop(0, _NCHUNK, body, 0, unroll=False)

    for b in range(_NBUF):
        pltpu.make_async_copy(
            rows_v.at[b], out_hbm.at[pl.ds(base, _CHUNK), pl.ds(0, HDIM)],
            w_sems.at[b],
        ).wait()


@jax.jit
def kernel(x, weight):
    idx = x.reshape(_NW * _NCHUNK, _CHUNK).astype(jnp.int32)
    table = weight.reshape(VOCAB_SIZE // 2, _PAD).reshape(VOCAB_SIZE, HDIM)
    mesh = plsc.VectorSubcoreMesh(core_axis_name="c", subcore_axis_name="s")
    out = pl.kernel(
        _embed_body,
        mesh=mesh,
        compiler_params=pltpu.CompilerParams(use_tc_tiling_on_sc=False),
        out_type=jax.ShapeDtypeStruct((_TOTAL, _PAD), jnp.float32),
        scratch_types=[
            pltpu.VMEM((_NCHUNK, _CHUNK), jnp.int32),
            pltpu.VMEM((_NBUF, _CHUNK, HDIM), jnp.float32),
            pltpu.SemaphoreType.DMA((_NBUF,)),
            pltpu.SemaphoreType.DMA((_NBUF,)),
        ],
    )(idx, table)
    return out[:, :HDIM].reshape(B, L, HDIM)
